# baseline (speedup 1.0000x reference)
import functools

import jax
import jax.numpy as jnp
from jax import lax
from jax.experimental import pallas as pl
from jax.experimental.pallas import tpu as pltpu
from jax.experimental.pallas import tpu_sc as plsc

N_NODES = 10000
N_EDGES = 320000
HID = 128
OUT_DIM = 10
N_LAYER = 3
N_GRAPHS = 128
EPS = 1e-5

NPAD = 10240
NW = 32
RPT = NPAD // 16
EPW = N_EDGES // NW
K = 80
NCH = EPW // K
NST = RPT // K

ROW_BLK = 1000
N_RB = N_NODES // ROW_BLK


@functools.cache
def _make_sc_degree():
    mesh = plsc.VectorSubcoreMesh(core_axis_name="c", subcore_axis_name="s")

    @functools.partial(
        pl.kernel,
        out_type=jax.ShapeDtypeStruct((2 * NPAD,), jnp.float32),
        mesh=mesh,
        scratch_types=[
            pltpu.VMEM((NCH, K), jnp.int32),
            pltpu.VMEM((K,), jnp.float32),
            pltpu.VMEM((RPT,), jnp.float32),
            pltpu.VMEM_SHARED((NPAD,), jnp.float32),
            pltpu.SemaphoreType.DMA,
        ],
    )
    def _sc_degree_impl(dst_hbm, zeros1_hbm, ones_hbm, out_hbm,
                        dst_a, ones_v, zbuf, acc, sem):
        cid = lax.axis_index("c")
        sid = lax.axis_index("s")
        wid = cid * 16 + sid
        pltpu.sync_copy(dst_hbm.at[wid], dst_a)
        pltpu.sync_copy(zeros1_hbm, zbuf)
        pltpu.sync_copy(zbuf, acc.at[pl.ds(sid * RPT, RPT)])
        pltpu.sync_copy(ones_hbm, ones_v)
        plsc.subcore_barrier()

        def body(j, carry):
            pltpu.async_copy(ones_v, acc.at[dst_a.at[j]], sem, add=True)
            return carry

        lax.fori_loop(0, NCH, body, 0)

        def drain(j, carry):
            pltpu.make_async_copy(ones_v, acc.at[dst_a.at[j]], sem).wait()
            return carry

        lax.fori_loop(0, NCH, drain, 0)
        plsc.subcore_barrier()
        pltpu.sync_copy(acc.at[pl.ds(sid * RPT, RPT)], zbuf)
        pltpu.sync_copy(zbuf,
                        out_hbm.at[pl.ds(cid * NPAD + sid * RPT, RPT)])

    return _sc_degree_impl


def _sc_degree(dst, zeros1, ones_k):
    return _make_sc_degree()(dst.reshape(NW, NCH, K), zeros1, ones_k)


@functools.cache
def _make_sc_scatter():
    mesh = plsc.VectorSubcoreMesh(core_axis_name="c", subcore_axis_name="s")

    @functools.partial(
        pl.kernel,
        out_type=jax.ShapeDtypeStruct((2, NPAD, HID), jnp.float32),
        mesh=mesh,
        scratch_types=[
            pltpu.VMEM((EPW,), jnp.int32),
            pltpu.VMEM((NCH, K), jnp.int32),
            pltpu.VMEM((K, HID), jnp.float32),
            pltpu.VMEM((K, HID), jnp.float32),
            pltpu.VMEM_SHARED((NPAD, HID), jnp.float32),
            pltpu.SemaphoreType.DMA,
            pltpu.SemaphoreType.DMA,
            pltpu.SemaphoreType.DMA,
            pltpu.SemaphoreType.DMA,
        ],
    )
    def _sc_scatter_impl(y_hbm, src_hbm, dst_hbm, zeros_hbm, out_hbm,
                         src_a, dst_a, rows0, rows1, acc,
                         gsem0, gsem1, ssem0, ssem1):
        cid = lax.axis_index("c")
        sid = lax.axis_index("s")
        wid = cid * 16 + sid
        pltpu.sync_copy(src_hbm.at[wid], src_a)
        pltpu.sync_copy(dst_hbm.at[wid], dst_a)
        pltpu.sync_copy(zeros_hbm, rows0)
        for t in range(NST):
            pltpu.sync_copy(rows0, acc.at[pl.ds(sid * RPT + t * K, K)])
        plsc.subcore_barrier()

        def g(j, buf, sem):
            pltpu.async_copy(y_hbm.at[src_a.at[pl.ds(j * K, K)]], buf, sem)

        def gwait(j, buf, sem):
            pltpu.make_async_copy(y_hbm.at[src_a.at[pl.ds(j * K, K)]],
                                  buf, sem).wait()

        def s(j, buf, sem):
            pltpu.async_copy(buf, acc.at[dst_a.at[j]], sem, add=True)

        def swait(j, buf, sem):
            pltpu.make_async_copy(buf, acc.at[dst_a.at[j]], sem).wait()

        g(0, rows0, gsem0)
        g(1, rows1, gsem1)

        def body(t, carry):
            j0 = 2 * t
            gwait(j0, rows0, gsem0)
            s(j0, rows0, ssem0)
            gwait(j0 + 1, rows1, gsem1)
            s(j0 + 1, rows1, ssem1)
            swait(j0, rows0, ssem0)
            g(j0 + 2, rows0, gsem0)
            swait(j0 + 1, rows1, ssem1)
            g(j0 + 3, rows1, gsem1)
            return carry

        lax.fori_loop(0, (NCH - 3) // 2, body, 0)
        jA = NCH - 3
        gwait(jA, rows0, gsem0)
        s(jA, rows0, ssem0)
        gwait(jA + 1, rows1, gsem1)
        s(jA + 1, rows1, ssem1)
        swait(jA, rows0, ssem0)
        g(jA + 2, rows0, gsem0)
        swait(jA + 1, rows1, ssem1)
        gwait(jA + 2, rows0, gsem0)
        s(jA + 2, rows0, ssem0)
        swait(jA + 2, rows0, ssem0)

        plsc.subcore_barrier()
        for t in range(NST):
            pltpu.sync_copy(acc.at[pl.ds(sid * RPT + t * K, K)], rows0)
            pltpu.sync_copy(rows0,
                            out_hbm.at[cid, pl.ds(sid * RPT + t * K, K)])

    return _sc_scatter_impl


def _sc_scatter(y, src, dst, zeros2):
    return _make_sc_scatter()(y, src.reshape(NW, EPW),
                              dst.reshape(NW, NCH, K), zeros2)


def _t1_body(deg0, deg1, x, W_emb, b_emb, Wc0, h0_ref, y_ref, dinv_ref):
    dinv = lax.rsqrt(deg0[...] + deg1[...] + 1.0)
    h0 = jnp.dot(x[...], W_emb[...],
                 preferred_element_type=jnp.float32) + b_emb[...]
    h0_ref[...] = h0
    y_ref[...] = jnp.dot(h0, Wc0[...],
                         preferred_element_type=jnp.float32) * dinv
    dinv_ref[...] = dinv


def _tc_embed(deg0, deg1, x, W_emb, b_emb, Wc0):
    rb = lambda i: (i, 0)
    cb = lambda i: (0, 0)
    return pl.pallas_call(
        _t1_body,
        grid=(N_RB,),
        in_specs=[
            pl.BlockSpec((ROW_BLK, 1), rb),
            pl.BlockSpec((ROW_BLK, 1), rb),
            pl.BlockSpec((ROW_BLK, HID), rb),
            pl.BlockSpec((HID, HID), cb),
            pl.BlockSpec((1, HID), cb),
            pl.BlockSpec((HID, HID), cb),
        ],
        out_specs=[
            pl.BlockSpec((ROW_BLK, HID), rb),
            pl.BlockSpec((ROW_BLK, HID), rb),
            pl.BlockSpec((ROW_BLK, 1), rb),
        ],
        out_shape=[
            jax.ShapeDtypeStruct((N_NODES, HID), jnp.float32),
            jax.ShapeDtypeStruct((N_NODES, HID), jnp.float32),
            jax.ShapeDtypeStruct((N_NODES, 1), jnp.float32),
        ],
    )(deg0, deg1, x, W_emb, b_emb, Wc0)


def _post_a_body(p0, p1, y, dinv, bci, s_ref, stats_ref):
    i = pl.program_id(0)

    @pl.when(i == 0)
    def _():
        stats_ref[...] = jnp.zeros_like(stats_ref)

    s = (p0[...] + p1[...] + y[...]) * dinv[...] + bci[...]
    s_ref[...] = s
    stats_ref[0:1, :] += jnp.sum(s, axis=0, keepdims=True)
    stats_ref[1:2, :] += jnp.sum(s * s, axis=0, keepdims=True)


def _tc_post_a(p0, p1, y, dinv, bci):
    rb = lambda i: (i, 0)
    cb = lambda i: (0, 0)
    return pl.pallas_call(
        _post_a_body,
        grid=(N_RB,),
        in_specs=[
            pl.BlockSpec((ROW_BLK, HID), rb),
            pl.BlockSpec((ROW_BLK, HID), rb),
            pl.BlockSpec((ROW_BLK, HID), rb),
            pl.BlockSpec((ROW_BLK, 1), rb),
            pl.BlockSpec((1, HID), cb),
        ],
        out_specs=[
            pl.BlockSpec((ROW_BLK, HID), rb),
            pl.BlockSpec((8, HID), cb),
        ],
        out_shape=[
            jax.ShapeDtypeStruct((N_NODES, HID), jnp.float32),
            jax.ShapeDtypeStruct((8, HID), jnp.float32),
        ],
    )(p0, p1, y, dinv, bci)


def _bn_relu_res(s, stats, g, b, h_old):
    n = jnp.float32(N_NODES)
    mu = stats[0:1, :] / n
    var = stats[1:2, :] / n - mu * mu
    z = g * (s - mu) * lax.rsqrt(var + EPS) + b
    return jnp.maximum(z, 0.0) + h_old


def _post_b_body(s, stats, g, b, h, dinv, Wn, h_ref, y_ref):
    h_new = _bn_relu_res(s[...], stats[...], g[...], b[...], h[...])
    h_ref[...] = h_new
    y_ref[...] = jnp.dot(h_new, Wn[...],
                         preferred_element_type=jnp.float32) * dinv[...]


def _tc_post_b(s, stats, g, b, h, dinv, Wn):
    rb = lambda i: (i, 0)
    cb = lambda i: (0, 0)
    return pl.pallas_call(
        _post_b_body,
        grid=(N_RB,),
        in_specs=[
            pl.BlockSpec((ROW_BLK, HID), rb),
            pl.BlockSpec((8, HID), cb),
            pl.BlockSpec((1, HID), cb),
            pl.BlockSpec((1, HID), cb),
            pl.BlockSpec((ROW_BLK, HID), rb),
            pl.BlockSpec((ROW_BLK, 1), rb),
            pl.BlockSpec((HID, HID), cb),
        ],
        out_specs=[
            pl.BlockSpec((ROW_BLK, HID), rb),
            pl.BlockSpec((ROW_BLK, HID), rb),
        ],
        out_shape=[
            jax.ShapeDtypeStruct((N_NODES, HID), jnp.float32),
            jax.ShapeDtypeStruct((N_NODES, HID), jnp.float32),
        ],
    )(s, stats, g, b, h, dinv, Wn)


def _pool_body(s, stats, g, b, h, batch, W1, b1, W2, b2, W3, b3,
               out_ref, sums_acc, cnt_acc):
    i = pl.program_id(0)

    @pl.when(i == 0)
    def _():
        sums_acc[...] = jnp.zeros_like(sums_acc)
        cnt_acc[...] = jnp.zeros_like(cnt_acc)

    h_new = _bn_relu_res(s[...], stats[...], g[...], b[...], h[...])
    gids = lax.broadcasted_iota(jnp.int32, (ROW_BLK, N_GRAPHS), 1)
    onehot = (batch[...] == gids).astype(jnp.float32)
    sums_acc[...] += lax.dot_general(
        onehot, h_new, (((0,), (0,)), ((), ())),
        preferred_element_type=jnp.float32)
    ones_col = jnp.ones((ROW_BLK, 1), jnp.float32)
    cnt_acc[...] += lax.dot_general(
        onehot, ones_col, (((0,), (0,)), ((), ())),
        preferred_element_type=jnp.float32)

    @pl.when(i == N_RB - 1)
    def _():
        pooled = sums_acc[...] / jnp.maximum(cnt_acc[...], 1.0)
        o = jnp.maximum(jnp.dot(pooled, W1[...],
                                preferred_element_type=jnp.float32) + b1[...], 0.0)
        o = jnp.maximum(jnp.dot(o, W2[...],
                                preferred_element_type=jnp.float32) + b2[...], 0.0)
        out_ref[...] = jnp.dot(o, W3[...],
                               preferred_element_type=jnp.float32) + b3[...]


def _tc_pool(s, stats, g, b, h, batch, W1, b1, W2, b2, W3, b3):
    rb = lambda i: (i, 0)
    cb = lambda i: (0, 0)
    return pl.pallas_call(
        _pool_body,
        grid=(N_RB,),
        in_specs=[
            pl.BlockSpec((ROW_BLK, HID), rb),
            pl.BlockSpec((8, HID), cb),
            pl.BlockSpec((1, HID), cb),
            pl.BlockSpec((1, HID), cb),
            pl.BlockSpec((ROW_BLK, HID), rb),
            pl.BlockSpec((ROW_BLK, 1), rb),
            pl.BlockSpec((HID, HID), cb),
            pl.BlockSpec((1, HID), cb),
            pl.BlockSpec((HID, HID), cb),
            pl.BlockSpec((1, HID), cb),
            pl.BlockSpec((HID, OUT_DIM), cb),
            pl.BlockSpec((1, OUT_DIM), cb),
        ],
        out_specs=pl.BlockSpec((N_GRAPHS, OUT_DIM), cb),
        out_shape=jax.ShapeDtypeStruct((N_GRAPHS, OUT_DIM), jnp.float32),
        scratch_shapes=[
            pltpu.VMEM((N_GRAPHS, HID), jnp.float32),
            pltpu.VMEM((N_GRAPHS, 1), jnp.float32),
        ],
    )(s, stats, g, b, h, batch, W1, b1, W2, b2, W3, b3)


def kernel(x, edge_index, cycle_index, batch, W_emb, b_emb, Wc, bc,
           gamma, beta, W1, b1, W2, b2, W3, b3):
    del cycle_index
    src = edge_index[0]
    dst = edge_index[1]
    zeros2 = jnp.zeros((K, HID), jnp.float32)
    zeros1 = jnp.zeros((RPT,), jnp.float32)
    ones_k = jnp.ones((K,), jnp.float32)

    deg_p = _sc_degree(dst, zeros1, ones_k)
    deg0 = deg_p[:N_NODES].reshape(N_NODES, 1)
    deg1 = deg_p[NPAD:NPAD + N_NODES].reshape(N_NODES, 1)

    h, y, dinv = _tc_embed(deg0, deg1, x, W_emb,
                           b_emb.reshape(1, HID), Wc[0])

    batch_col = batch.reshape(N_NODES, 1)
    for i in range(N_LAYER):
        p = _sc_scatter(y, src, dst, zeros2)
        s, stats = _tc_post_a(p[0, :N_NODES], p[1, :N_NODES], y, dinv,
                              bc[i].reshape(1, HID))
        gi = gamma[i].reshape(1, HID)
        bi = beta[i].reshape(1, HID)
        if i < N_LAYER - 1:
            h, y = _tc_post_b(s, stats, gi, bi, h, dinv, Wc[i + 1])
        else:
            out = _tc_pool(s, stats, gi, bi, h, batch_col,
                           W1, b1.reshape(1, HID), W2, b2.reshape(1, HID),
                           W3, b3.reshape(1, OUT_DIM))
    return out


# speedup vs baseline: 1.3002x; 1.3002x over previous
import functools

import jax
import jax.numpy as jnp
from jax import lax
from jax.experimental import pallas as pl
from jax.experimental.pallas import tpu as pltpu
from jax.experimental.pallas import tpu_sc as plsc

N_NODES = 10000
N_EDGES = 320000
HID = 128
OUT_DIM = 10
N_LAYER = 3
N_GRAPHS = 128
EPS = 1e-5

NPAD = 10240
NW = 32
RPT = NPAD // 16
EPW = N_EDGES // NW
K = 80
NCH = EPW // K
NST = RPT // K

ROW_BLK = 1000
N_RB = N_NODES // ROW_BLK


@functools.cache
def _make_sc_degree():
    mesh = plsc.VectorSubcoreMesh(core_axis_name="c", subcore_axis_name="s")

    @functools.partial(
        pl.kernel,
        out_type=jax.ShapeDtypeStruct((2 * NPAD,), jnp.float32),
        mesh=mesh,
        scratch_types=[
            pltpu.VMEM((NCH, K), jnp.int32),
            pltpu.VMEM((K,), jnp.float32),
            pltpu.VMEM((RPT,), jnp.float32),
            pltpu.VMEM_SHARED((NPAD,), jnp.float32),
            pltpu.SemaphoreType.DMA,
        ],
    )
    def _sc_degree_impl(dst_hbm, zeros1_hbm, ones_hbm, out_hbm,
                        dst_a, ones_v, zbuf, acc, sem):
        cid = lax.axis_index("c")
        sid = lax.axis_index("s")
        wid = cid * 16 + sid
        pltpu.sync_copy(dst_hbm.at[wid], dst_a)
        pltpu.sync_copy(zeros1_hbm, zbuf)
        pltpu.sync_copy(zbuf, acc.at[pl.ds(sid * RPT, RPT)])
        pltpu.sync_copy(ones_hbm, ones_v)
        plsc.subcore_barrier()

        def body(j, carry):
            pltpu.async_copy(ones_v, acc.at[dst_a.at[j]], sem, add=True)
            return carry

        lax.fori_loop(0, NCH, body, 0)

        def drain(j, carry):
            pltpu.make_async_copy(ones_v, acc.at[dst_a.at[j]], sem).wait()
            return carry

        lax.fori_loop(0, NCH, drain, 0)
        plsc.subcore_barrier()
        pltpu.sync_copy(acc.at[pl.ds(sid * RPT, RPT)], zbuf)
        pltpu.sync_copy(zbuf,
                        out_hbm.at[pl.ds(cid * NPAD + sid * RPT, RPT)])

    return _sc_degree_impl


def _sc_degree(dst, zeros1, ones_k):
    return _make_sc_degree()(dst.reshape(NW, NCH, K), zeros1, ones_k)


@functools.cache
def _make_sc_scatter():
    mesh = plsc.VectorSubcoreMesh(core_axis_name="c", subcore_axis_name="s")

    @functools.partial(
        pl.kernel,
        out_type=jax.ShapeDtypeStruct((2, NPAD, HID), jnp.float32),
        mesh=mesh,
        scratch_types=[
            pltpu.VMEM((EPW,), jnp.int32),
            pltpu.VMEM((NCH, K), jnp.int32),
            pltpu.VMEM((K, HID), jnp.float32),
            pltpu.VMEM((K, HID), jnp.float32),
            pltpu.VMEM_SHARED((NPAD, HID), jnp.float32),
            pltpu.SemaphoreType.DMA,
            pltpu.SemaphoreType.DMA,
            pltpu.SemaphoreType.DMA,
            pltpu.SemaphoreType.DMA,
        ],
    )
    def _sc_scatter_impl(y_hbm, src_hbm, dst_hbm, zeros_hbm, out_hbm,
                         src_a, dst_a, rows0, rows1, acc,
                         gsem0, gsem1, ssem0, ssem1):
        cid = lax.axis_index("c")
        sid = lax.axis_index("s")
        wid = cid * 16 + sid
        pltpu.sync_copy(src_hbm.at[wid], src_a)
        pltpu.sync_copy(dst_hbm.at[wid], dst_a)
        pltpu.sync_copy(zeros_hbm, rows0)
        for t in range(NST):
            pltpu.sync_copy(rows0, acc.at[pl.ds(sid * RPT + t * K, K)])
        plsc.subcore_barrier()

        def g(j, buf, sem):
            pltpu.async_copy(y_hbm.at[src_a.at[pl.ds(j * K, K)]], buf, sem)

        def gwait(j, buf, sem):
            pltpu.make_async_copy(y_hbm.at[src_a.at[pl.ds(j * K, K)]],
                                  buf, sem).wait()

        g(0, rows0, gsem0)

        def body(t, carry):
            j0 = 2 * t
            g(j0 + 1, rows1, gsem1)
            gwait(j0, rows0, gsem0)
            pltpu.sync_copy(rows0, acc.at[dst_a.at[j0]], add=True)
            g(j0 + 2, rows0, gsem0)
            gwait(j0 + 1, rows1, gsem1)
            pltpu.sync_copy(rows1, acc.at[dst_a.at[j0 + 1]], add=True)
            return carry

        lax.fori_loop(0, (NCH - 1) // 2, body, 0)
        gwait(NCH - 1, rows0, gsem0)
        pltpu.sync_copy(rows0, acc.at[dst_a.at[NCH - 1]], add=True)

        plsc.subcore_barrier()
        for t in range(NST):
            pltpu.sync_copy(acc.at[pl.ds(sid * RPT + t * K, K)], rows0)
            pltpu.sync_copy(rows0,
                            out_hbm.at[cid, pl.ds(sid * RPT + t * K, K)])

    return _sc_scatter_impl


def _sc_scatter(y, src, dst, zeros2):
    return _make_sc_scatter()(y, src.reshape(NW, EPW),
                              dst.reshape(NW, NCH, K), zeros2)


def _t1_body(deg0, deg1, x, W_emb, b_emb, Wc0, h0_ref, y_ref, dinv_ref):
    dinv = lax.rsqrt(deg0[...] + deg1[...] + 1.0)
    h0 = jnp.dot(x[...], W_emb[...],
                 preferred_element_type=jnp.float32) + b_emb[...]
    h0_ref[...] = h0
    y_ref[...] = jnp.dot(h0, Wc0[...],
                         preferred_element_type=jnp.float32) * dinv
    dinv_ref[...] = dinv


def _tc_embed(deg0, deg1, x, W_emb, b_emb, Wc0):
    rb = lambda i: (i, 0)
    cb = lambda i: (0, 0)
    return pl.pallas_call(
        _t1_body,
        grid=(N_RB,),
        in_specs=[
            pl.BlockSpec((ROW_BLK, 1), rb),
            pl.BlockSpec((ROW_BLK, 1), rb),
            pl.BlockSpec((ROW_BLK, HID), rb),
            pl.BlockSpec((HID, HID), cb),
            pl.BlockSpec((1, HID), cb),
            pl.BlockSpec((HID, HID), cb),
        ],
        out_specs=[
            pl.BlockSpec((ROW_BLK, HID), rb),
            pl.BlockSpec((ROW_BLK, HID), rb),
            pl.BlockSpec((ROW_BLK, 1), rb),
        ],
        out_shape=[
            jax.ShapeDtypeStruct((N_NODES, HID), jnp.float32),
            jax.ShapeDtypeStruct((N_NODES, HID), jnp.float32),
            jax.ShapeDtypeStruct((N_NODES, 1), jnp.float32),
        ],
    )(deg0, deg1, x, W_emb, b_emb, Wc0)


def _post_a_body(p0, p1, y, dinv, bci, s_ref, stats_ref):
    i = pl.program_id(0)

    @pl.when(i == 0)
    def _():
        stats_ref[...] = jnp.zeros_like(stats_ref)

    s = (p0[...] + p1[...] + y[...]) * dinv[...] + bci[...]
    s_ref[...] = s
    stats_ref[0:1, :] += jnp.sum(s, axis=0, keepdims=True)
    stats_ref[1:2, :] += jnp.sum(s * s, axis=0, keepdims=True)


def _tc_post_a(p0, p1, y, dinv, bci):
    rb = lambda i: (i, 0)
    cb = lambda i: (0, 0)
    return pl.pallas_call(
        _post_a_body,
        grid=(N_RB,),
        in_specs=[
            pl.BlockSpec((ROW_BLK, HID), rb),
            pl.BlockSpec((ROW_BLK, HID), rb),
            pl.BlockSpec((ROW_BLK, HID), rb),
            pl.BlockSpec((ROW_BLK, 1), rb),
            pl.BlockSpec((1, HID), cb),
        ],
        out_specs=[
            pl.BlockSpec((ROW_BLK, HID), rb),
            pl.BlockSpec((8, HID), cb),
        ],
        out_shape=[
            jax.ShapeDtypeStruct((N_NODES, HID), jnp.float32),
            jax.ShapeDtypeStruct((8, HID), jnp.float32),
        ],
    )(p0, p1, y, dinv, bci)


def _bn_relu_res(s, stats, g, b, h_old):
    n = jnp.float32(N_NODES)
    mu = stats[0:1, :] / n
    var = stats[1:2, :] / n - mu * mu
    z = g * (s - mu) * lax.rsqrt(var + EPS) + b
    return jnp.maximum(z, 0.0) + h_old


def _post_b_body(s, stats, g, b, h, dinv, Wn, h_ref, y_ref):
    h_new = _bn_relu_res(s[...], stats[...], g[...], b[...], h[...])
    h_ref[...] = h_new
    y_ref[...] = jnp.dot(h_new, Wn[...],
                         preferred_element_type=jnp.float32) * dinv[...]


def _tc_post_b(s, stats, g, b, h, dinv, Wn):
    rb = lambda i: (i, 0)
    cb = lambda i: (0, 0)
    return pl.pallas_call(
        _post_b_body,
        grid=(N_RB,),
        in_specs=[
            pl.BlockSpec((ROW_BLK, HID), rb),
            pl.BlockSpec((8, HID), cb),
            pl.BlockSpec((1, HID), cb),
            pl.BlockSpec((1, HID), cb),
            pl.BlockSpec((ROW_BLK, HID), rb),
            pl.BlockSpec((ROW_BLK, 1), rb),
            pl.BlockSpec((HID, HID), cb),
        ],
        out_specs=[
            pl.BlockSpec((ROW_BLK, HID), rb),
            pl.BlockSpec((ROW_BLK, HID), rb),
        ],
        out_shape=[
            jax.ShapeDtypeStruct((N_NODES, HID), jnp.float32),
            jax.ShapeDtypeStruct((N_NODES, HID), jnp.float32),
        ],
    )(s, stats, g, b, h, dinv, Wn)


def _pool_body(s, stats, g, b, h, batch, W1, b1, W2, b2, W3, b3,
               out_ref, sums_acc, cnt_acc):
    i = pl.program_id(0)

    @pl.when(i == 0)
    def _():
        sums_acc[...] = jnp.zeros_like(sums_acc)
        cnt_acc[...] = jnp.zeros_like(cnt_acc)

    h_new = _bn_relu_res(s[...], stats[...], g[...], b[...], h[...])
    gids = lax.broadcasted_iota(jnp.int32, (ROW_BLK, N_GRAPHS), 1)
    onehot = (batch[...] == gids).astype(jnp.float32)
    sums_acc[...] += lax.dot_general(
        onehot, h_new, (((0,), (0,)), ((), ())),
        preferred_element_type=jnp.float32)
    ones_col = jnp.ones((ROW_BLK, 1), jnp.float32)
    cnt_acc[...] += lax.dot_general(
        onehot, ones_col, (((0,), (0,)), ((), ())),
        preferred_element_type=jnp.float32)

    @pl.when(i == N_RB - 1)
    def _():
        pooled = sums_acc[...] / jnp.maximum(cnt_acc[...], 1.0)
        o = jnp.maximum(jnp.dot(pooled, W1[...],
                                preferred_element_type=jnp.float32) + b1[...], 0.0)
        o = jnp.maximum(jnp.dot(o, W2[...],
                                preferred_element_type=jnp.float32) + b2[...], 0.0)
        out_ref[...] = jnp.dot(o, W3[...],
                               preferred_element_type=jnp.float32) + b3[...]


def _tc_pool(s, stats, g, b, h, batch, W1, b1, W2, b2, W3, b3):
    rb = lambda i: (i, 0)
    cb = lambda i: (0, 0)
    return pl.pallas_call(
        _pool_body,
        grid=(N_RB,),
        in_specs=[
            pl.BlockSpec((ROW_BLK, HID), rb),
            pl.BlockSpec((8, HID), cb),
            pl.BlockSpec((1, HID), cb),
            pl.BlockSpec((1, HID), cb),
            pl.BlockSpec((ROW_BLK, HID), rb),
            pl.BlockSpec((ROW_BLK, 1), rb),
            pl.BlockSpec((HID, HID), cb),
            pl.BlockSpec((1, HID), cb),
            pl.BlockSpec((HID, HID), cb),
            pl.BlockSpec((1, HID), cb),
            pl.BlockSpec((HID, OUT_DIM), cb),
            pl.BlockSpec((1, OUT_DIM), cb),
        ],
        out_specs=pl.BlockSpec((N_GRAPHS, OUT_DIM), cb),
        out_shape=jax.ShapeDtypeStruct((N_GRAPHS, OUT_DIM), jnp.float32),
        scratch_shapes=[
            pltpu.VMEM((N_GRAPHS, HID), jnp.float32),
            pltpu.VMEM((N_GRAPHS, 1), jnp.float32),
        ],
    )(s, stats, g, b, h, batch, W1, b1, W2, b2, W3, b3)


def kernel(x, edge_index, cycle_index, batch, W_emb, b_emb, Wc, bc,
           gamma, beta, W1, b1, W2, b2, W3, b3):
    del cycle_index
    src = edge_index[0]
    dst = edge_index[1]
    zeros2 = jnp.zeros((K, HID), jnp.float32)
    zeros1 = jnp.zeros((RPT,), jnp.float32)
    ones_k = jnp.ones((K,), jnp.float32)

    deg_p = _sc_degree(dst, zeros1, ones_k)
    deg0 = deg_p[:N_NODES].reshape(N_NODES, 1)
    deg1 = deg_p[NPAD:NPAD + N_NODES].reshape(N_NODES, 1)

    h, y, dinv = _tc_embed(deg0, deg1, x, W_emb,
                           b_emb.reshape(1, HID), Wc[0])

    batch_col = batch.reshape(N_NODES, 1)
    for i in range(N_LAYER):
        p = _sc_scatter(y, src, dst, zeros2)
        s, stats = _tc_post_a(p[0, :N_NODES], p[1, :N_NODES], y, dinv,
                              bc[i].reshape(1, HID))
        gi = gamma[i].reshape(1, HID)
        bi = beta[i].reshape(1, HID)
        if i < N_LAYER - 1:
            h, y = _tc_post_b(s, stats, gi, bi, h, dinv, Wc[i + 1])
        else:
            out = _tc_pool(s, stats, gi, bi, h, batch_col,
                           W1, b1.reshape(1, HID), W2, b2.reshape(1, HID),
                           W3, b3.reshape(1, OUT_DIM))
    return out
